# baseline (speedup 1.0000x reference)
import functools

import jax
import jax.numpy as jnp
from jax import lax
from jax.experimental import pallas as pl
from jax.experimental.pallas import tpu as pltpu, tpu_sc as plsc

N_NODES = 10000
N_EDGES = 320000
IN_DIM = 128
HID_DIM = 256
OUT_DIM = 128
WEIGHT_DECAY = 1e-4

NC, NS = 2, 16
NW = NC * NS
NPT = 313
NPT_PAD = 320
CAP = 16384
CH = 3200
B = 128
TB = 2048

_mesh = plsc.VectorSubcoreMesh(core_axis_name="c", subcore_axis_name="s")
_CP = pltpu.CompilerParams(needs_layout_passes=False)
_f32 = jnp.float32
_i32 = jnp.int32


@functools.partial(
    pl.kernel,
    out_type=(
        jax.ShapeDtypeStruct((NW, NPT_PAD), _f32),
        jax.ShapeDtypeStruct((NW, CAP), _i32),
        jax.ShapeDtypeStruct((NW, CAP), _i32),
        jax.ShapeDtypeStruct((NW, CAP), _i32),
        jax.ShapeDtypeStruct((NW, CAP), _i32),
        jax.ShapeDtypeStruct((NW, 16), _i32),
        jax.ShapeDtypeStruct((NW, 16), _i32),
    ),
    mesh=_mesh,
    compiler_params=_CP,
    scratch_types=[
        pltpu.VMEM((2, CH), _i32),
        pltpu.VMEM((2, CH), _i32),
        pltpu.VMEM((CAP,), _i32),
        pltpu.VMEM((CAP,), _i32),
        pltpu.VMEM((CAP,), _i32),
        pltpu.VMEM((CAP,), _i32),
        pltpu.VMEM((NPT_PAD,), _f32),
        pltpu.VMEM((16,), _i32),
        pltpu.SemaphoreType.DMA,
        pltpu.SemaphoreType.DMA,
    ],
)
def _scan_kernel(src_hbm, dst_hbm, dinv_hbm, l1s_hbm, l1d_hbm, l2s_hbm,
                 l2d_hbm, c1_hbm, c2_hbm,
                 sv_ref, dv_ref, l1s, l1d, l2s, l2d, degv, cntv, semA, semB):
    cid = lax.axis_index("c")
    sid = lax.axis_index("s")
    wid = sid * NC + cid
    lo = wid * NPT
    hi = lo + NPT

    zi = jnp.zeros((16,), _i32)
    zf = jnp.zeros((16,), _f32)

    def zero_lists(i):
        l1s[pl.ds(i * 16, 16)] = zi
        l1d[pl.ds(i * 16, 16)] = zi
        l2s[pl.ds(i * 16, 16)] = zi
        l2d[pl.ds(i * 16, 16)] = zi

    plsc.parallel_loop(0, CAP // 16, step=1)(zero_lists)

    def zero_deg(i, _):
        degv[pl.ds(i * 16, 16)] = zf
        return 0

    lax.fori_loop(0, NPT_PAD // 16, zero_deg, 0)

    ones = jnp.ones((16,), _f32)
    NOUT = N_EDGES // CH
    UNROLL = 16

    def fetch_chunk(o, buf, sem):
        base = o * CH
        pltpu.async_copy(src_hbm.at[pl.ds(base, CH)], sv_ref.at[buf], sem)
        pltpu.async_copy(dst_hbm.at[pl.ds(base, CH)], dv_ref.at[buf], sem)

    def wait_chunk(buf, sem):
        pltpu.make_async_copy(src_hbm.at[pl.ds(0, CH)], sv_ref.at[buf],
                              sem).wait()
        pltpu.make_async_copy(dst_hbm.at[pl.ds(0, CH)], dv_ref.at[buf],
                              sem).wait()

    def scan_chunk(buf, carry):
        def inner(g4, c):
            q1, q2 = c
            for u in range(UNROLL):
                g = g4 * UNROLL + u
                sv = sv_ref[buf, pl.ds(g * 16, 16)]
                dv = dv_ref[buf, pl.ds(g * 16, 16)]
                m1 = (dv >= lo) & (dv < hi)
                dl = dv - lo
                plsc.store_compressed(l1s.at[pl.ds(q1, 16)], sv, mask=m1)
                plsc.store_compressed(l1d.at[pl.ds(q1, 16)], dl, mask=m1)
                q1 = q1 + plsc.all_reduce_population_count(m1)[0]
                m2 = (sv >= lo) & (sv < hi)
                sl = sv - lo
                plsc.store_compressed(l2s.at[pl.ds(q2, 16)], sl, mask=m2)
                plsc.store_compressed(l2d.at[pl.ds(q2, 16)], dv, mask=m2)
                q2 = q2 + plsc.all_reduce_population_count(m2)[0]
            return (q1, q2)

        return plsc.parallel_loop(0, CH // (16 * UNROLL), step=1,
                                  carry=carry)(inner)

    fetch_chunk(0, 0, semA)

    def pair(p, carry):
        fetch_chunk(2 * p + 1, 1, semB)
        wait_chunk(0, semA)
        carry = scan_chunk(0, carry)

        @pl.when(p + 1 < NOUT // 2)
        def _():
            fetch_chunk(2 * p + 2, 0, semA)

        wait_chunk(1, semB)
        return scan_chunk(1, carry)

    p1, p2 = lax.fori_loop(0, NOUT // 2, pair,
                           (jnp.int32(0), jnp.int32(0)))

    iota16 = lax.iota(_i32, 16)
    n1g = (p1 + 15) // 16

    def degp(g):
        dl = l1d[pl.ds(g * 16, 16)]
        m = (g * 16 + iota16) < p1
        plsc.addupdate_scatter(degv, [dl], ones, mask=m)

    plsc.parallel_loop(0, n1g, step=1)(degp)

    def newton(i, _):
        d = degv[pl.ds(i * 16, 16)] + 1.0
        bi = plsc.bitcast(d, _i32)
        y = plsc.bitcast(jnp.int32(0x5F3759DF) - (bi >> 1), _f32)
        y = y * (1.5 - 0.5 * d * y * y)
        y = y * (1.5 - 0.5 * d * y * y)
        y = y * (1.5 - 0.5 * d * y * y)
        degv[pl.ds(i * 16, 16)] = y
        return 0

    lax.fori_loop(0, NPT_PAD // 16, newton, 0)

    pltpu.sync_copy(degv, dinv_hbm.at[wid])
    pltpu.sync_copy(l1s, l1s_hbm.at[wid])
    pltpu.sync_copy(l1d, l1d_hbm.at[wid])
    pltpu.sync_copy(l2s, l2s_hbm.at[wid])
    pltpu.sync_copy(l2d, l2d_hbm.at[wid])
    cntv[...] = jnp.broadcast_to(p1, (16,))
    pltpu.sync_copy(cntv, c1_hbm.at[wid])
    cntv[...] = jnp.broadcast_to(p2, (16,))
    pltpu.sync_copy(cntv, c2_hbm.at[wid])


@functools.partial(
    pl.kernel,
    out_type=(
        jax.ShapeDtypeStruct((NW, NPT, IN_DIM), _f32),
        jax.ShapeDtypeStruct((NW, NPT_PAD), _f32),
    ),
    mesh=_mesh,
    compiler_params=_CP,
    scratch_types=[
        pltpu.VMEM((N_NODES,), _f32),
        pltpu.VMEM((2, TB), _i32),
        pltpu.VMEM((2, TB), _i32),
        pltpu.VMEM((2, B, IN_DIM), _f32),
        pltpu.VMEM((NPT, IN_DIM), _f32),
        pltpu.VMEM((NPT_PAD,), _f32),
        pltpu.VMEM((TB,), _i32),
        pltpu.VMEM((TB,), _i32),
        pltpu.VMEM((16,), _i32),
        pltpu.SemaphoreType.DMA,
        pltpu.SemaphoreType.DMA,
        pltpu.SemaphoreType.DMA,
        pltpu.SemaphoreType.DMA,
    ],
)
def _agg_kernel(x_hbm, dinv_hbm, l1s_hbm, l1d_hbm, l2s_hbm, l2d_hbm,
                c1_hbm, c2_hbm, agg_hbm, t_hbm,
                dinv_v, is_v, id_v, rows, aggv, tv, t2s, t2d, cntv,
                sem0, sem1, semL0, semL1):
    cid = lax.axis_index("c")
    sid = lax.axis_index("s")
    wid = sid * NC + cid

    pltpu.sync_copy(dinv_hbm, dinv_v)

    zf = jnp.zeros((16,), _f32)

    def zero_agg(i):
        for k in range(IN_DIM // 16):
            aggv[i, pl.ds(k * 16, 16)] = zf

    plsc.parallel_loop(0, NPT, step=1)(zero_agg)

    def zero_t(i, _):
        tv[pl.ds(i * 16, 16)] = zf
        return 0

    lax.fori_loop(0, NPT_PAD // 16, zero_t, 0)

    iota = lax.iota(_i32, 16)

    pltpu.sync_copy(c1_hbm.at[wid], cntv)
    c1 = cntv[pl.ds(0, 16)][0]
    nbig = (c1 + (TB - 1)) // TB
    NSUB = TB // B

    def fetch_big(bi, bbuf, semL):
        base = bi * TB
        pltpu.async_copy(l1s_hbm.at[wid, pl.ds(base, TB)], is_v.at[bbuf], semL)
        pltpu.async_copy(l1d_hbm.at[wid, pl.ds(base, TB)], id_v.at[bbuf], semL)

    def wait_big(bbuf, semL):
        pltpu.make_async_copy(l1s_hbm.at[wid, pl.ds(0, TB)], is_v.at[bbuf],
                              semL).wait()
        pltpu.make_async_copy(l1d_hbm.at[wid, pl.ds(0, TB)], id_v.at[bbuf],
                              semL).wait()

    def fetch_rows(bbuf, sub, rbuf, sem):
        idx = is_v.at[bbuf, pl.ds(sub * B, B)]
        pltpu.async_copy(x_hbm.at[idx], rows.at[rbuf], sem)

    def wait_rows(bbuf, rbuf, sem):
        pltpu.make_async_copy(x_hbm.at[is_v.at[bbuf, pl.ds(0, B)]],
                              rows.at[rbuf], sem).wait()

    def process(bi, bbuf, sub, rbuf):
        boff = sub * B
        base = bi * TB + boff

        def grp(g):
            sv = is_v[bbuf, pl.ds(boff + g * 16, 16)]
            dlv = id_v[bbuf, pl.ds(boff + g * 16, 16)]
            dsc = plsc.load_gather(dinv_v, [sv])
            m = (base + g * 16 + iota) < c1
            dsc = jnp.where(m, dsc, 0.0)
            for e in range(16):
                lane = jnp.full((16,), e, _i32)
                dlb = dlv.at[lane].get(mode="promise_in_bounds")
                db = dsc.at[lane].get(mode="promise_in_bounds")
                row = g * 16 + e
                for k in range(IN_DIM // 16):
                    vals = rows[rbuf, row, pl.ds(k * 16, 16)] * db
                    plsc.addupdate_scatter(aggv, [dlb, k * 16 + iota], vals)

        plsc.parallel_loop(0, B // 16, step=1, unroll=4)(grp)

    def process_big(bi, bbuf):
        rem = c1 - bi * TB
        nsub = jnp.minimum((rem + (B - 1)) // B, NSUB)
        fetch_rows(bbuf, 0, 0, sem0)

        def subpair(q, _):
            s0 = 2 * q
            s1 = s0 + 1

            @pl.when(s1 < nsub)
            def _():
                fetch_rows(bbuf, s1, 1, sem1)

            wait_rows(bbuf, 0, sem0)
            process(bi, bbuf, s0, 0)

            @pl.when(s0 + 2 < nsub)
            def _():
                fetch_rows(bbuf, s0 + 2, 0, sem0)

            @pl.when(s1 < nsub)
            def _():
                wait_rows(bbuf, 1, sem1)
                process(bi, bbuf, s1, 1)

            return 0

        lax.fori_loop(0, (nsub + 1) // 2, subpair, 0)

    @pl.when(nbig > 0)
    def _():
        fetch_big(0, 0, semL0)

        def bigpair(p, _):
            b0 = 2 * p
            b1 = b0 + 1

            @pl.when(b1 < nbig)
            def _():
                fetch_big(b1, 1, semL1)

            wait_big(0, semL0)
            process_big(b0, 0)

            @pl.when(b0 + 2 < nbig)
            def _():
                fetch_big(b0 + 2, 0, semL0)

            @pl.when(b1 < nbig)
            def _():
                wait_big(1, semL1)
                process_big(b1, 1)

            return 0

        lax.fori_loop(0, (nbig + 1) // 2, bigpair, 0)

    pltpu.sync_copy(c2_hbm.at[wid], cntv)
    c2 = cntv[pl.ds(0, 16)][0]
    nch2 = (c2 + (TB - 1)) // TB

    def chunk2(ch, _):
        base = ch * TB
        pltpu.sync_copy(l2s_hbm.at[wid, pl.ds(base, TB)], t2s)
        pltpu.sync_copy(l2d_hbm.at[wid, pl.ds(base, TB)], t2d)

        def grp(g):
            slv = t2s[pl.ds(g * 16, 16)]
            dgv = t2d[pl.ds(g * 16, 16)]
            dd = plsc.load_gather(dinv_v, [dgv])
            m = (base + g * 16 + iota) < c2
            plsc.addupdate_scatter(tv, [slv], dd, mask=m)

        plsc.parallel_loop(0, TB // 16, step=1)(grp)
        return 0

    lax.fori_loop(0, nch2, chunk2, 0)

    pltpu.sync_copy(aggv, agg_hbm.at[wid])
    pltpu.sync_copy(tv, t_hbm.at[wid])


_R = 1000


def _dense_body(agg_ref, x_ref, dinv_ref, t_ref, w1_ref, b1_ref, w2_ref,
                b2_ref, out_ref, vacc):
    i = pl.program_id(0)

    @pl.when(i == 0)
    def _():
        vacc[...] = jnp.zeros_like(vacc)

    dinv = dinv_ref[...]
    pre = dinv * agg_ref[...] + (dinv * dinv) * x_ref[...]
    h = lax.dot_general(pre, w1_ref[...], (((1,), (1,)), ((), ())),
                        preferred_element_type=_f32)
    h = jnp.maximum(h + b1_ref[...], 0.0)
    c = dinv * (dinv + t_ref[...])
    vacc[...] += lax.dot_general(c, h, (((0,), (0,)), ((), ())),
                                 preferred_element_type=_f32)

    @pl.when(i == pl.num_programs(0) - 1)
    def _():
        v = vacc[...]
        o = lax.dot_general(v, w2_ref[...], (((1,), (1,)), ((), ())),
                            preferred_element_type=_f32)
        w1 = w1_ref[...]
        w2 = w2_ref[...]
        b1 = b1_ref[...]
        b2 = b2_ref[...]
        l2 = WEIGHT_DECAY * (jnp.sum(w1 * w1) + jnp.sum(b1 * b1)
                             + jnp.sum(w2 * w2) + jnp.sum(b2 * b2))
        out_ref[...] = o * _f32(1.0 / N_NODES) + b2 - 0.5 * l2


def _dense(agg, x, dinv2, t2, W1, b1, W2, b2):
    return pl.pallas_call(
        _dense_body,
        grid=(N_NODES // _R,),
        in_specs=[
            pl.BlockSpec((_R, IN_DIM), lambda i: (i, 0)),
            pl.BlockSpec((_R, IN_DIM), lambda i: (i, 0)),
            pl.BlockSpec((_R, 1), lambda i: (i, 0)),
            pl.BlockSpec((_R, 1), lambda i: (i, 0)),
            pl.BlockSpec((HID_DIM, IN_DIM), lambda i: (0, 0)),
            pl.BlockSpec((1, HID_DIM), lambda i: (0, 0)),
            pl.BlockSpec((OUT_DIM, HID_DIM), lambda i: (0, 0)),
            pl.BlockSpec((1, OUT_DIM), lambda i: (0, 0)),
        ],
        out_specs=pl.BlockSpec((1, OUT_DIM), lambda i: (0, 0)),
        out_shape=jax.ShapeDtypeStruct((1, OUT_DIM), _f32),
        scratch_shapes=[pltpu.VMEM((1, HID_DIM), _f32)],
    )(agg, x, dinv2, t2, W1, b1, W2, b2)


def kernel(x, edge_index, W1, b1, W2, b2):
    src = edge_index[0].astype(_i32)
    dst = edge_index[1].astype(_i32)

    dinv_p, l1s, l1d, l2s, l2d, c1, c2 = _scan_kernel(src, dst)
    dinv = dinv_p[:, :NPT].reshape(-1)[:N_NODES]

    agg_p, t_p = _agg_kernel(x, dinv, l1s, l1d, l2s, l2d, c1, c2)
    agg = agg_p.reshape(NW * NPT, IN_DIM)[:N_NODES]
    t = t_p[:, :NPT].reshape(-1)[:N_NODES]

    out = _dense(agg, x, dinv[:, None], t[:, None],
                 W1, b1[None, :], W2, b2[None, :])
    return out.reshape(OUT_DIM)


# speedup vs baseline: 1.4196x; 1.4196x over previous
import functools

import jax
import jax.numpy as jnp
from jax import lax
from jax.experimental import pallas as pl
from jax.experimental.pallas import tpu as pltpu, tpu_sc as plsc

N_NODES = 10000
N_EDGES = 320000
IN_DIM = 128
HID_DIM = 256
OUT_DIM = 128
WEIGHT_DECAY = 1e-4

NC, NS = 2, 16
NW = NC * NS
NPT = 313
NPT_PAD = 320
CAP = 16384
CH = 3200
B = 128
TB = 2048

_mesh = plsc.VectorSubcoreMesh(core_axis_name="c", subcore_axis_name="s")
_CP = pltpu.CompilerParams(needs_layout_passes=False)
_f32 = jnp.float32
_i32 = jnp.int32


@functools.partial(
    pl.kernel,
    out_type=(
        jax.ShapeDtypeStruct((NW, NPT_PAD), _f32),
        jax.ShapeDtypeStruct((NW, CAP), _i32),
        jax.ShapeDtypeStruct((NW, CAP), _i32),
        jax.ShapeDtypeStruct((NW, CAP), _i32),
        jax.ShapeDtypeStruct((NW, CAP), _i32),
        jax.ShapeDtypeStruct((NW, 16), _i32),
        jax.ShapeDtypeStruct((NW, 16), _i32),
    ),
    mesh=_mesh,
    compiler_params=_CP,
    scratch_types=[
        pltpu.VMEM((2, CH), _i32),
        pltpu.VMEM((2, CH), _i32),
        pltpu.VMEM((CAP,), _i32),
        pltpu.VMEM((CAP,), _i32),
        pltpu.VMEM((CAP,), _i32),
        pltpu.VMEM((CAP,), _i32),
        pltpu.VMEM((NPT_PAD,), _f32),
        pltpu.VMEM((16,), _i32),
        pltpu.SemaphoreType.DMA,
        pltpu.SemaphoreType.DMA,
    ],
)
def _scan_kernel(src_hbm, dst_hbm, dinv_hbm, l1s_hbm, l1d_hbm, l2s_hbm,
                 l2d_hbm, c1_hbm, c2_hbm,
                 sv_ref, dv_ref, l1s, l1d, l2s, l2d, degv, cntv, semA, semB):
    cid = lax.axis_index("c")
    sid = lax.axis_index("s")
    wid = sid * NC + cid
    lo = wid * NPT
    hi = lo + NPT

    zi = jnp.zeros((16,), _i32)
    zf = jnp.zeros((16,), _f32)

    def zero_lists(i):
        l1s[pl.ds(i * 16, 16)] = zi
        l1d[pl.ds(i * 16, 16)] = zi
        l2s[pl.ds(i * 16, 16)] = zi
        l2d[pl.ds(i * 16, 16)] = zi

    plsc.parallel_loop(0, CAP // 16, step=1)(zero_lists)

    def zero_deg(i, _):
        degv[pl.ds(i * 16, 16)] = zf
        return 0

    lax.fori_loop(0, NPT_PAD // 16, zero_deg, 0)

    ones = jnp.ones((16,), _f32)
    NOUT = N_EDGES // CH
    UNROLL = 8

    def fetch_chunk(o, buf, sem):
        base = o * CH
        pltpu.async_copy(src_hbm.at[pl.ds(base, CH)], sv_ref.at[buf], sem)
        pltpu.async_copy(dst_hbm.at[pl.ds(base, CH)], dv_ref.at[buf], sem)

    def wait_chunk(buf, sem):
        pltpu.make_async_copy(src_hbm.at[pl.ds(0, CH)], sv_ref.at[buf],
                              sem).wait()
        pltpu.make_async_copy(dst_hbm.at[pl.ds(0, CH)], dv_ref.at[buf],
                              sem).wait()

    def scan_chunk(buf, carry):
        def inner(g4, c):
            q1, q2 = c
            for u in range(UNROLL):
                g = g4 * UNROLL + u
                sv = sv_ref[buf, pl.ds(g * 16, 16)]
                dv = dv_ref[buf, pl.ds(g * 16, 16)]
                m1 = (dv >= lo) & (dv < hi)
                dl = dv - lo
                plsc.store_compressed(l1s.at[pl.ds(q1, 16)], sv, mask=m1)
                plsc.store_compressed(l1d.at[pl.ds(q1, 16)], dl, mask=m1)
                q1 = q1 + plsc.all_reduce_population_count(m1)[0]
                m2 = (sv >= lo) & (sv < hi)
                sl = sv - lo
                plsc.store_compressed(l2s.at[pl.ds(q2, 16)], sl, mask=m2)
                plsc.store_compressed(l2d.at[pl.ds(q2, 16)], dv, mask=m2)
                q2 = q2 + plsc.all_reduce_population_count(m2)[0]
            return (q1, q2)

        return plsc.parallel_loop(0, CH // (16 * UNROLL), step=1,
                                  carry=carry)(inner)

    fetch_chunk(0, 0, semA)

    def pair(p, carry):
        fetch_chunk(2 * p + 1, 1, semB)
        wait_chunk(0, semA)
        carry = scan_chunk(0, carry)

        @pl.when(p + 1 < NOUT // 2)
        def _():
            fetch_chunk(2 * p + 2, 0, semA)

        wait_chunk(1, semB)
        return scan_chunk(1, carry)

    p1, p2 = lax.fori_loop(0, NOUT // 2, pair,
                           (jnp.int32(0), jnp.int32(0)))

    iota16 = lax.iota(_i32, 16)
    n1g = (p1 + 15) // 16

    def degp(g):
        dl = l1d[pl.ds(g * 16, 16)]
        m = (g * 16 + iota16) < p1
        plsc.addupdate_scatter(degv, [dl], ones, mask=m)

    plsc.parallel_loop(0, n1g, step=1)(degp)

    def newton(i, _):
        d = degv[pl.ds(i * 16, 16)] + 1.0
        bi = plsc.bitcast(d, _i32)
        y = plsc.bitcast(jnp.int32(0x5F3759DF) - (bi >> 1), _f32)
        y = y * (1.5 - 0.5 * d * y * y)
        y = y * (1.5 - 0.5 * d * y * y)
        y = y * (1.5 - 0.5 * d * y * y)
        degv[pl.ds(i * 16, 16)] = y
        return 0

    lax.fori_loop(0, NPT_PAD // 16, newton, 0)

    pltpu.sync_copy(degv, dinv_hbm.at[wid])
    pltpu.sync_copy(l1s, l1s_hbm.at[wid])
    pltpu.sync_copy(l1d, l1d_hbm.at[wid])
    pltpu.sync_copy(l2s, l2s_hbm.at[wid])
    pltpu.sync_copy(l2d, l2d_hbm.at[wid])
    cntv[...] = jnp.broadcast_to(p1, (16,))
    pltpu.sync_copy(cntv, c1_hbm.at[wid])
    cntv[...] = jnp.broadcast_to(p2, (16,))
    pltpu.sync_copy(cntv, c2_hbm.at[wid])


@functools.partial(
    pl.kernel,
    out_type=(
        jax.ShapeDtypeStruct((NW, NPT, IN_DIM), _f32),
        jax.ShapeDtypeStruct((NW, NPT_PAD), _f32),
    ),
    mesh=_mesh,
    compiler_params=_CP,
    scratch_types=[
        pltpu.VMEM((N_NODES,), _f32),
        pltpu.VMEM((2, TB), _i32),
        pltpu.VMEM((2, TB), _i32),
        pltpu.VMEM((2, B, IN_DIM), _f32),
        pltpu.VMEM((NPT, IN_DIM), _f32),
        pltpu.VMEM((NPT_PAD,), _f32),
        pltpu.VMEM((TB,), _i32),
        pltpu.VMEM((TB,), _i32),
        pltpu.VMEM((16,), _i32),
        pltpu.SemaphoreType.DMA,
        pltpu.SemaphoreType.DMA,
        pltpu.SemaphoreType.DMA,
        pltpu.SemaphoreType.DMA,
    ],
)
def _agg_kernel(x_hbm, dinv_hbm, l1s_hbm, l1d_hbm, l2s_hbm, l2d_hbm,
                c1_hbm, c2_hbm, agg_hbm, t_hbm,
                dinv_v, is_v, id_v, rows, aggv, tv, t2s, t2d, cntv,
                sem0, sem1, semL0, semL1):
    cid = lax.axis_index("c")
    sid = lax.axis_index("s")
    wid = sid * NC + cid

    pltpu.sync_copy(dinv_hbm, dinv_v)

    zf = jnp.zeros((16,), _f32)

    def zero_agg(i):
        for k in range(IN_DIM // 16):
            aggv[i, pl.ds(k * 16, 16)] = zf

    plsc.parallel_loop(0, NPT, step=1)(zero_agg)

    def zero_t(i, _):
        tv[pl.ds(i * 16, 16)] = zf
        return 0

    lax.fori_loop(0, NPT_PAD // 16, zero_t, 0)

    iota = lax.iota(_i32, 16)

    pltpu.sync_copy(c1_hbm.at[wid], cntv)
    c1 = cntv[pl.ds(0, 16)][0]
    nbig = (c1 + (TB - 1)) // TB
    NSUB = TB // B

    def fetch_big(bi, bbuf, semL):
        base = bi * TB
        pltpu.async_copy(l1s_hbm.at[wid, pl.ds(base, TB)], is_v.at[bbuf], semL)
        pltpu.async_copy(l1d_hbm.at[wid, pl.ds(base, TB)], id_v.at[bbuf], semL)

    def wait_big(bbuf, semL):
        pltpu.make_async_copy(l1s_hbm.at[wid, pl.ds(0, TB)], is_v.at[bbuf],
                              semL).wait()
        pltpu.make_async_copy(l1d_hbm.at[wid, pl.ds(0, TB)], id_v.at[bbuf],
                              semL).wait()

    def fetch_rows(bbuf, sub, rbuf, sem):
        idx = is_v.at[bbuf, pl.ds(sub * B, B)]
        pltpu.async_copy(x_hbm.at[idx], rows.at[rbuf], sem)

    def wait_rows(bbuf, rbuf, sem):
        pltpu.make_async_copy(x_hbm.at[is_v.at[bbuf, pl.ds(0, B)]],
                              rows.at[rbuf], sem).wait()

    def process(bi, bbuf, sub, rbuf):
        boff = sub * B
        base = bi * TB + boff

        def grp(g):
            sv = is_v[bbuf, pl.ds(boff + g * 16, 16)]
            dlv = id_v[bbuf, pl.ds(boff + g * 16, 16)]
            dsc = plsc.load_gather(dinv_v, [sv])
            m = (base + g * 16 + iota) < c1
            dsc = jnp.where(m, dsc, 0.0)
            for e in range(16):
                lane = jnp.full((16,), e, _i32)
                dlb = dlv.at[lane].get(mode="promise_in_bounds")
                db = dsc.at[lane].get(mode="promise_in_bounds")
                row = g * 16 + e
                for k in range(IN_DIM // 16):
                    vals = rows[rbuf, row, pl.ds(k * 16, 16)] * db
                    plsc.addupdate_scatter(aggv, [dlb, k * 16 + iota], vals)

        plsc.parallel_loop(0, B // 16, step=1, unroll=2)(grp)

    def process_big(bi, bbuf):
        rem = c1 - bi * TB
        nsub = jnp.minimum((rem + (B - 1)) // B, NSUB)
        fetch_rows(bbuf, 0, 0, sem0)

        def subpair(q, _):
            s0 = 2 * q
            s1 = s0 + 1

            @pl.when(s1 < nsub)
            def _():
                fetch_rows(bbuf, s1, 1, sem1)

            wait_rows(bbuf, 0, sem0)
            process(bi, bbuf, s0, 0)

            @pl.when(s0 + 2 < nsub)
            def _():
                fetch_rows(bbuf, s0 + 2, 0, sem0)

            @pl.when(s1 < nsub)
            def _():
                wait_rows(bbuf, 1, sem1)
                process(bi, bbuf, s1, 1)

            return 0

        lax.fori_loop(0, (nsub + 1) // 2, subpair, 0)

    @pl.when(nbig > 0)
    def _():
        fetch_big(0, 0, semL0)

        def bigpair(p, _):
            b0 = 2 * p
            b1 = b0 + 1

            @pl.when(b1 < nbig)
            def _():
                fetch_big(b1, 1, semL1)

            wait_big(0, semL0)
            process_big(b0, 0)

            @pl.when(b0 + 2 < nbig)
            def _():
                fetch_big(b0 + 2, 0, semL0)

            @pl.when(b1 < nbig)
            def _():
                wait_big(1, semL1)
                process_big(b1, 1)

            return 0

        lax.fori_loop(0, (nbig + 1) // 2, bigpair, 0)

    pltpu.sync_copy(c2_hbm.at[wid], cntv)
    c2 = cntv[pl.ds(0, 16)][0]
    nch2 = (c2 + (TB - 1)) // TB

    def chunk2(ch, _):
        base = ch * TB
        pltpu.sync_copy(l2s_hbm.at[wid, pl.ds(base, TB)], t2s)
        pltpu.sync_copy(l2d_hbm.at[wid, pl.ds(base, TB)], t2d)

        def grp(g):
            slv = t2s[pl.ds(g * 16, 16)]
            dgv = t2d[pl.ds(g * 16, 16)]
            dd = plsc.load_gather(dinv_v, [dgv])
            m = (base + g * 16 + iota) < c2
            plsc.addupdate_scatter(tv, [slv], dd, mask=m)

        plsc.parallel_loop(0, TB // 16, step=1)(grp)
        return 0

    lax.fori_loop(0, nch2, chunk2, 0)

    pltpu.sync_copy(aggv, agg_hbm.at[wid])
    pltpu.sync_copy(tv, t_hbm.at[wid])


_R = 1000


def _dense_body(agg_ref, x_ref, dinv_ref, t_ref, w1_ref, b1_ref, w2_ref,
                b2_ref, out_ref, vacc):
    i = pl.program_id(0)

    @pl.when(i == 0)
    def _():
        vacc[...] = jnp.zeros_like(vacc)

    dinv = dinv_ref[...]
    pre = dinv * agg_ref[...] + (dinv * dinv) * x_ref[...]
    h = lax.dot_general(pre, w1_ref[...], (((1,), (1,)), ((), ())),
                        preferred_element_type=_f32)
    h = jnp.maximum(h + b1_ref[...], 0.0)
    c = dinv * (dinv + t_ref[...])
    vacc[...] += lax.dot_general(c, h, (((0,), (0,)), ((), ())),
                                 preferred_element_type=_f32)

    @pl.when(i == pl.num_programs(0) - 1)
    def _():
        v = vacc[...]
        o = lax.dot_general(v, w2_ref[...], (((1,), (1,)), ((), ())),
                            preferred_element_type=_f32)
        w1 = w1_ref[...]
        w2 = w2_ref[...]
        b1 = b1_ref[...]
        b2 = b2_ref[...]
        l2 = WEIGHT_DECAY * (jnp.sum(w1 * w1) + jnp.sum(b1 * b1)
                             + jnp.sum(w2 * w2) + jnp.sum(b2 * b2))
        out_ref[...] = o * _f32(1.0 / N_NODES) + b2 - 0.5 * l2


def _dense(agg, x, dinv2, t2, W1, b1, W2, b2):
    return pl.pallas_call(
        _dense_body,
        grid=(N_NODES // _R,),
        in_specs=[
            pl.BlockSpec((_R, IN_DIM), lambda i: (i, 0)),
            pl.BlockSpec((_R, IN_DIM), lambda i: (i, 0)),
            pl.BlockSpec((_R, 1), lambda i: (i, 0)),
            pl.BlockSpec((_R, 1), lambda i: (i, 0)),
            pl.BlockSpec((HID_DIM, IN_DIM), lambda i: (0, 0)),
            pl.BlockSpec((1, HID_DIM), lambda i: (0, 0)),
            pl.BlockSpec((OUT_DIM, HID_DIM), lambda i: (0, 0)),
            pl.BlockSpec((1, OUT_DIM), lambda i: (0, 0)),
        ],
        out_specs=pl.BlockSpec((1, OUT_DIM), lambda i: (0, 0)),
        out_shape=jax.ShapeDtypeStruct((1, OUT_DIM), _f32),
        scratch_shapes=[pltpu.VMEM((1, HID_DIM), _f32)],
    )(agg, x, dinv2, t2, W1, b1, W2, b2)


def kernel(x, edge_index, W1, b1, W2, b2):
    src = edge_index[0].astype(_i32)
    dst = edge_index[1].astype(_i32)

    dinv_p, l1s, l1d, l2s, l2d, c1, c2 = _scan_kernel(src, dst)
    dinv = dinv_p[:, :NPT].reshape(-1)[:N_NODES]

    agg_p, t_p = _agg_kernel(x, dinv, l1s, l1d, l2s, l2d, c1, c2)
    agg = agg_p.reshape(NW * NPT, IN_DIM)[:N_NODES]
    t = t_p[:, :NPT].reshape(-1)[:N_NODES]

    out = _dense(agg, x, dinv[:, None], t[:, None],
                 W1, b1[None, :], W2, b2[None, :])
    return out.reshape(OUT_DIM)
